# baseline (speedup 1.0000x reference)
import jax
import jax.numpy as jnp
from jax import lax
from jax.experimental import pallas as pl
from jax.experimental.pallas import tpu as pltpu
from jax.experimental.pallas import tpu_sc as plsc

_RATE = 0.1
_KEEP = 1.0 - _RATE
_SEED = 42

_K0 = 0
_K1 = _SEED
_K2 = _K0 ^ _K1 ^ 0x1BD11BDA

_ROT_A = (13, 15, 26, 6)
_ROT_B = (17, 29, 16, 24)
_KS = (_K0, _K1, _K2)

_THRESH = 7549747 << 9

_NNZ = 2684354

_NW = 32
_SC_BLK = 4096
_SC_NBLK = 7
_SC_UNROLL = 4
_SPLIT = _NW * _SC_NBLK * _SC_BLK

_CHUNK = 65536
assert _SPLIT % _CHUNK == 0


def _threefry_bits(lo):
    x0 = jnp.full_like(lo, jnp.uint32(_K0))
    x1 = lo + jnp.uint32(_K1)
    for i in range(5):
        rots = _ROT_A if i % 2 == 0 else _ROT_B
        for r in rots:
            x0 = x0 + x1
            x1 = (x1 << jnp.uint32(r)) | (x1 >> jnp.uint32(32 - r))
            x1 = x1 ^ x0
        x0 = x0 + jnp.uint32(_KS[(i + 1) % 3])
        x1 = x1 + jnp.uint32(_KS[(i + 2) % 3] + i + 1)
    return x0 ^ x1


def _tc_body(v_ref, o_ref):
    pid = pl.program_id(0)
    rows, cols = _CHUNK // 1024, 1024
    row = lax.broadcasted_iota(jnp.uint32, (rows, cols), 0)
    col = lax.broadcasted_iota(jnp.uint32, (rows, cols), 1)
    base = jnp.uint32(_SPLIT) + jnp.uint32(pid) * jnp.uint32(_CHUNK)
    idx = row * jnp.uint32(cols) + col + base
    bits = _threefry_bits(idx)
    keep = bits < jnp.uint32(_THRESH)
    v2 = v_ref[...].reshape(rows, cols)
    out = jnp.where(keep, v2 / jnp.float32(_KEEP), jnp.float32(0.0))
    o_ref[...] = out.reshape(_CHUNK)


def _tc_part(values):
    nblk = pl.cdiv(_NNZ - _SPLIT, _CHUNK)
    off = _SPLIT // _CHUNK
    return pl.pallas_call(
        _tc_body,
        grid=(nblk,),
        in_specs=[pl.BlockSpec((_CHUNK,), lambda i: (i + off,))],
        out_specs=pl.BlockSpec((_CHUNK,), lambda i: (i + off,)),
        out_shape=jax.ShapeDtypeStruct((_NNZ,), jnp.float32),
        compiler_params=pltpu.CompilerParams(
            dimension_semantics=("arbitrary",)),
    )(values)


def _sc_fn(v_hbm, o_hbm, vin, vout):
    wid = lax.axis_index("s") * 2 + lax.axis_index("c")
    base = wid * (_SC_NBLK * _SC_BLK)
    lane = lax.broadcasted_iota(jnp.uint32, (16,), 0)

    def blk_body(b, carry):
        off = base + b * _SC_BLK
        pltpu.sync_copy(v_hbm.at[pl.ds(off, _SC_BLK)], vin)
        off32 = lax.convert_element_type(off, jnp.uint32)

        def vec_body(j, c):
            for u in range(_SC_UNROLL):
                e = j * (16 * _SC_UNROLL) + u * 16
                lo = lane + (off32 + jnp.uint32(e))
                bits = _threefry_bits(lo)
                keep = bits < jnp.uint32(_THRESH)
                vals = vin[pl.ds(e, 16)]
                vout[pl.ds(e, 16)] = jnp.where(
                    keep, vals / jnp.float32(_KEEP), jnp.float32(0.0))
            return c

        lax.fori_loop(0, _SC_BLK // (16 * _SC_UNROLL), vec_body, 0)
        pltpu.sync_copy(vout, o_hbm.at[pl.ds(off, _SC_BLK)])
        return carry

    lax.fori_loop(0, _SC_NBLK, blk_body, 0)


def _sc_part(values):
    mesh = plsc.VectorSubcoreMesh(core_axis_name="c", subcore_axis_name="s")
    run = pl.kernel(
        _sc_fn,
        mesh=mesh,
        out_type=jax.ShapeDtypeStruct((_SPLIT,), jnp.float32),
        scratch_types=[
            pltpu.VMEM((_SC_BLK,), jnp.float32),
            pltpu.VMEM((_SC_BLK,), jnp.float32),
        ],
    )
    return run(values)


def kernel(values, indices):
    sc_out = _sc_part(values)
    tc_out = _tc_part(values)
    return lax.dynamic_update_slice(tc_out, sc_out, (0,)), indices


# speedup vs baseline: 1.1659x; 1.1659x over previous
import jax
import jax.numpy as jnp
from jax import lax
from jax.experimental import pallas as pl
from jax.experimental.pallas import tpu as pltpu
from jax.experimental.pallas import tpu_sc as plsc

_RATE = 0.1
_KEEP = 1.0 - _RATE
_SEED = 42

_K0 = 0
_K1 = _SEED
_K2 = _K0 ^ _K1 ^ 0x1BD11BDA

_ROT_A = (13, 15, 26, 6)
_ROT_B = (17, 29, 16, 24)
_KS = (_K0, _K1, _K2)

_THRESH = 7549747 << 9

_NNZ = 2684354

_NW = 32
_SC_BLK = 4096
_SC_NBLK = 3
_SC_UNROLL = 4
_SPLIT = _NW * _SC_NBLK * _SC_BLK

_CHUNK = 65536
assert _SPLIT % _CHUNK == 0


def _threefry_bits(lo):
    x0 = jnp.full_like(lo, jnp.uint32(_K0))
    x1 = lo + jnp.uint32(_K1)
    for i in range(5):
        rots = _ROT_A if i % 2 == 0 else _ROT_B
        for r in rots:
            x0 = x0 + x1
            x1 = (x1 << jnp.uint32(r)) | (x1 >> jnp.uint32(32 - r))
            x1 = x1 ^ x0
        x0 = x0 + jnp.uint32(_KS[(i + 1) % 3])
        x1 = x1 + jnp.uint32(_KS[(i + 2) % 3] + i + 1)
    return x0 ^ x1


def _tc_body(v_ref, o_ref):
    pid = pl.program_id(0)
    rows, cols = _CHUNK // 1024, 1024
    row = lax.broadcasted_iota(jnp.uint32, (rows, cols), 0)
    col = lax.broadcasted_iota(jnp.uint32, (rows, cols), 1)
    base = jnp.uint32(_SPLIT) + jnp.uint32(pid) * jnp.uint32(_CHUNK)
    idx = row * jnp.uint32(cols) + col + base
    bits = _threefry_bits(idx)
    keep = bits < jnp.uint32(_THRESH)
    v2 = v_ref[...].reshape(rows, cols)
    out = jnp.where(keep, v2 / jnp.float32(_KEEP), jnp.float32(0.0))
    o_ref[...] = out.reshape(_CHUNK)


def _tc_part(values):
    nblk = pl.cdiv(_NNZ - _SPLIT, _CHUNK)
    off = _SPLIT // _CHUNK
    return pl.pallas_call(
        _tc_body,
        grid=(nblk,),
        in_specs=[pl.BlockSpec((_CHUNK,), lambda i: (i + off,))],
        out_specs=pl.BlockSpec((_CHUNK,), lambda i: (i + off,)),
        out_shape=jax.ShapeDtypeStruct((_NNZ,), jnp.float32),
        compiler_params=pltpu.CompilerParams(
            dimension_semantics=("arbitrary",)),
    )(values)


def _sc_fn(v_hbm, o_hbm, vin, vout):
    wid = lax.axis_index("s") * 2 + lax.axis_index("c")
    base = wid * (_SC_NBLK * _SC_BLK)
    lane = lax.broadcasted_iota(jnp.uint32, (16,), 0)

    def blk_body(b, carry):
        off = base + b * _SC_BLK
        pltpu.sync_copy(v_hbm.at[pl.ds(off, _SC_BLK)], vin)
        off32 = lax.convert_element_type(off, jnp.uint32)

        def vec_body(j, c):
            for u in range(_SC_UNROLL):
                e = j * (16 * _SC_UNROLL) + u * 16
                lo = lane + (off32 + jnp.uint32(e))
                bits = _threefry_bits(lo)
                keep = bits < jnp.uint32(_THRESH)
                vals = vin[pl.ds(e, 16)]
                vout[pl.ds(e, 16)] = jnp.where(
                    keep, vals / jnp.float32(_KEEP), jnp.float32(0.0))
            return c

        lax.fori_loop(0, _SC_BLK // (16 * _SC_UNROLL), vec_body, 0)
        pltpu.sync_copy(vout, o_hbm.at[pl.ds(off, _SC_BLK)])
        return carry

    lax.fori_loop(0, _SC_NBLK, blk_body, 0)


def _sc_part(values):
    mesh = plsc.VectorSubcoreMesh(core_axis_name="c", subcore_axis_name="s")
    run = pl.kernel(
        _sc_fn,
        mesh=mesh,
        out_type=jax.ShapeDtypeStruct((_SPLIT,), jnp.float32),
        scratch_types=[
            pltpu.VMEM((_SC_BLK,), jnp.float32),
            pltpu.VMEM((_SC_BLK,), jnp.float32),
        ],
    )
    return run(values)


def kernel(values, indices):
    sc_out = _sc_part(values)
    tc_out = _tc_part(values)
    return lax.dynamic_update_slice(tc_out, sc_out, (0,)), indices
